# Initial kernel scaffold; baseline (speedup 1.0000x reference)
import jax
import jax.numpy as jnp
from jax.experimental import pallas as pl


def kernel(cam_indices, pt_indices, base_pose_indices, relative_pose_indices, is_relative, pixels_measured, base_quat, base_trans, rel_quat, rel_trans, points, intrinsics):
    raise NotImplementedError("write your pallas kernel here")


# speedup vs baseline: 12.5211x; 12.5211x over previous
import functools

import jax
import jax.numpy as jnp
from jax import lax
from jax.experimental import pallas as pl
from jax.experimental.pallas import tpu as pltpu
from jax.experimental.pallas import tpu_sc as plsc

_NC = 2
_NS = 16
_NW = _NC * _NS
_L = 16
_CHUNK = 1024


def _quat_planes(w, x, y, z):
    return [
        1.0 - 2.0 * (y * y + z * z), 2.0 * (x * y - w * z), 2.0 * (x * z + w * y),
        2.0 * (x * y + w * z), 1.0 - 2.0 * (x * x + z * z), 2.0 * (y * z - w * x),
        2.0 * (x * z - w * y), 2.0 * (y * z + w * x), 1.0 - 2.0 * (x * x + y * y),
    ]


def _table_body(bq_ref, bt_ref, rq_ref, rt_ref, k_ref, base_ref, rel_ref):
    ncam = k_ref.shape[0] // 9

    w = bq_ref[0:1, :]
    x = bq_ref[1:2, :]
    y = bq_ref[2:3, :]
    z = bq_ref[3:4, :]
    inv = lax.rsqrt(w * w + x * x + y * y + z * z)
    w, x, y, z = w * inv, x * inv, y * inv, z * inv
    rb = _quat_planes(w, x, y, z)

    wr = rq_ref[:, 0:1]
    xr = rq_ref[:, 1:2]
    yr = rq_ref[:, 2:3]
    zr = rq_ref[:, 3:4]
    invr = lax.rsqrt(wr * wr + xr * xr + yr * yr + zr * zr)
    wr, xr, yr, zr = wr * invr, xr * invr, yr * invr, zr * invr
    rr = _quat_planes(wr, xr, yr, zr)

    rc = [None] * 9
    for i in range(3):
        for l in range(3):
            acc = rr[0 + l] * rb[3 * i + 0]
            acc = acc + rr[3 + l] * rb[3 * i + 1]
            acc = acc + rr[6 + l] * rb[3 * i + 2]
            rc[3 * i + l] = acc

    bt = [bt_ref[i:i + 1, :] for i in range(3)]
    rt = [rt_ref[:, i:i + 1] for i in range(3)]
    tc = [None] * 3
    for i in range(3):
        acc = bt[i] + rc[3 * i + 0] * rt[0]
        acc = acc + rc[3 * i + 1] * rt[1]
        acc = acc + rc[3 * i + 2] * rt[2]
        tc[i] = acc

    for c in range(ncam):
        kk = [k_ref[9 * c + e] for e in range(9)]
        for i in range(3):
            for l in range(3):
                row = c * 12 + 3 * i + l
                mb = kk[3 * i] * rb[l] + kk[3 * i + 1] * rb[3 + l] + kk[3 * i + 2] * rb[6 + l]
                base_ref[row:row + 1, :] = mb
                mr = kk[3 * i] * rc[l] + kk[3 * i + 1] * rc[3 + l] + kk[3 * i + 2] * rc[6 + l]
                rel_ref[row, :, :] = mr
            row = c * 12 + 9 + i
            vb = kk[3 * i] * bt[0] + kk[3 * i + 1] * bt[1] + kk[3 * i + 2] * bt[2]
            base_ref[row:row + 1, :] = vb
            vr = kk[3 * i] * tc[0] + kk[3 * i + 1] * tc[1] + kk[3 * i + 2] * tc[2]
            rel_ref[row, :, :] = vr


def _build_table(base_quat, base_trans, rel_quat, rel_trans, intrinsics):
    nb = base_quat.shape[0]
    nr = rel_quat.shape[0]
    ncam = intrinsics.shape[0]
    base_p, rel_p = pl.pallas_call(
        _table_body,
        out_shape=[
            jax.ShapeDtypeStruct((ncam * 12, nb), jnp.float32),
            jax.ShapeDtypeStruct((ncam * 12, nr, nb), jnp.float32),
        ],
        in_specs=[
            pl.BlockSpec(memory_space=pltpu.VMEM),
            pl.BlockSpec(memory_space=pltpu.VMEM),
            pl.BlockSpec(memory_space=pltpu.VMEM),
            pl.BlockSpec(memory_space=pltpu.VMEM),
            pl.BlockSpec(memory_space=pltpu.SMEM),
        ],
    )(base_quat.T, base_trans.T, rel_quat, rel_trans,
      intrinsics.reshape(ncam * 9))
    tb = base_p.reshape(ncam, 12, nb).transpose(0, 2, 1)
    tr = rel_p.reshape(ncam, 12, nr, nb).transpose(0, 2, 3, 1)
    full = jnp.concatenate([tb, tr.reshape(ncam, nr * nb, 12)], axis=1)
    full = jnp.pad(full, ((0, 0), (0, 0), (0, 4)))
    return full.reshape(ncam * (1 + nr) * nb, 16)


def _make_sc_kernel(npad, nb, nr, w_per):
    jrows = (1 + nr) * nb
    n_chunks = w_per // _CHUNK
    C = _CHUNK

    @functools.partial(
        pl.kernel,
        out_type=jax.ShapeDtypeStruct((2 * npad,), jnp.float32),
        mesh=plsc.VectorSubcoreMesh(core_axis_name="c", subcore_axis_name="s",
                                    num_cores=_NC, num_subcores=_NS),
        compiler_params=pltpu.CompilerParams(
            needs_layout_passes=False, use_tc_tiling_on_sc=False),
        scratch_types=[
            pltpu.VMEM((C,), jnp.int32),
            pltpu.VMEM((C,), jnp.int32),
            pltpu.VMEM((C,), jnp.int32),
            pltpu.VMEM((C,), jnp.int32),
            pltpu.VMEM((C,), jnp.int32),
            pltpu.VMEM((C,), jnp.int32),
            pltpu.VMEM((2 * C,), jnp.float32),
            pltpu.VMEM((C, 16), jnp.float32),
            pltpu.VMEM((C, 16), jnp.float32),
            pltpu.VMEM((2 * C,), jnp.float32),
            pltpu.SemaphoreType.DMA,
            pltpu.SemaphoreType.DMA,
        ],
    )
    def sc_kernel(b_hbm, r_hbm, m_hbm, c_hbm, p_hbm, meas_hbm, table_hbm,
                  pts_hbm, out_hbm, b_v, r_v, m_v, c_v, p_v, j_v, meas_v,
                  rows_v, pts_v, out_v, sem_a, sem_b):
        wid = lax.axis_index("s") * _NC + lax.axis_index("c")
        base = wid * w_per
        iota = lax.iota(jnp.int32, _L)

        def chunk_body(g, _):
            off = base + g * C
            pltpu.sync_copy(b_hbm.at[pl.ds(off, C)], b_v)
            pltpu.sync_copy(r_hbm.at[pl.ds(off, C)], r_v)
            pltpu.sync_copy(m_hbm.at[pl.ds(off, C)], m_v)
            pltpu.sync_copy(c_hbm.at[pl.ds(off, C)], c_v)
            pltpu.sync_copy(p_hbm.at[pl.ds(off, C)], p_v)
            pltpu.sync_copy(meas_hbm.at[pl.ds(2 * off, 2 * C)], meas_v)

            def j_body(v, _):
                o = v * _L
                bb = b_v[pl.ds(o, _L)]
                rr = r_v[pl.ds(o, _L)]
                mm = m_v[pl.ds(o, _L)]
                cc = c_v[pl.ds(o, _L)]
                j = jnp.where(mm > 0, (rr + 1) * nb + bb, bb)
                j_v[pl.ds(o, _L)] = j + cc * jrows
                return 0

            lax.fori_loop(0, C // _L, j_body, 0)

            cp_rows = pltpu.make_async_copy(table_hbm.at[j_v], rows_v, sem_a)
            cp_rows.start()
            cp_pts = pltpu.make_async_copy(pts_hbm.at[p_v], pts_v, sem_b)
            cp_pts.start()
            cp_rows.wait()
            cp_pts.wait()

            def c_body(v, _):
                o = v * _L
                row = o + iota
                m = [plsc.load_gather(rows_v, [row, jnp.full((_L,), k, jnp.int32)])
                     for k in range(12)]
                px = plsc.load_gather(pts_v, [row, jnp.full((_L,), 0, jnp.int32)])
                py = plsc.load_gather(pts_v, [row, jnp.full((_L,), 1, jnp.int32)])
                pz = plsc.load_gather(pts_v, [row, jnp.full((_L,), 2, jnp.int32)])
                xn = m[0] * px + m[1] * py + m[2] * pz + m[9]
                yn = m[3] * px + m[4] * py + m[5] * pz + m[10]
                zn = m[6] * px + m[7] * py + m[8] * pz + m[11]
                inv = 1.0 / zn
                mx = plsc.load_gather(meas_v, [2 * row])
                my = plsc.load_gather(meas_v, [2 * row + 1])
                plsc.store_scatter(out_v, [2 * row], xn * inv - mx)
                plsc.store_scatter(out_v, [2 * row + 1], yn * inv - my)
                return 0

            lax.fori_loop(0, C // _L, c_body, 0)
            pltpu.sync_copy(out_v, out_hbm.at[pl.ds(2 * off, 2 * C)])
            return 0

        lax.fori_loop(0, n_chunks, chunk_body, 0)

    return sc_kernel


def kernel(cam_indices, pt_indices, base_pose_indices, relative_pose_indices,
           is_relative, pixels_measured, base_quat, base_trans, rel_quat,
           rel_trans, points, intrinsics):
    n = cam_indices.shape[0]
    nb = base_quat.shape[0]
    nr = rel_quat.shape[0]

    table = _build_table(base_quat, base_trans, rel_quat, rel_trans, intrinsics)
    pts_pad = jnp.pad(points, ((0, 0), (0, 13)))

    per = -(-n // (_NW * _CHUNK)) * _CHUNK
    npad = per * _NW
    pad = npad - n

    def pad1(a):
        return jnp.pad(a, (0, pad))

    b_i = pad1(base_pose_indices.astype(jnp.int32))
    r_i = pad1(relative_pose_indices.astype(jnp.int32))
    m_i = pad1(is_relative.astype(jnp.int32))
    c_i = pad1(cam_indices.astype(jnp.int32))
    p_i = pad1(pt_indices.astype(jnp.int32))
    meas = jnp.pad(pixels_measured.reshape(-1), (0, 2 * pad))

    sc = _make_sc_kernel(npad, nb, nr, per)
    out = sc(b_i, r_i, m_i, c_i, p_i, meas, table, pts_pad)
    return out[:2 * n]


# speedup vs baseline: 12.5246x; 1.0003x over previous
import functools

import jax
import jax.numpy as jnp
from jax import lax
from jax.experimental import pallas as pl
from jax.experimental.pallas import tpu as pltpu
from jax.experimental.pallas import tpu_sc as plsc

_NC = 2
_NS = 16
_NW = _NC * _NS
_L = 16
_CHUNK = 512


def _quat_planes(w, x, y, z):
    return [
        1.0 - 2.0 * (y * y + z * z), 2.0 * (x * y - w * z), 2.0 * (x * z + w * y),
        2.0 * (x * y + w * z), 1.0 - 2.0 * (x * x + z * z), 2.0 * (y * z - w * x),
        2.0 * (x * z - w * y), 2.0 * (y * z + w * x), 1.0 - 2.0 * (x * x + y * y),
    ]


def _table_body(bq_ref, bt_ref, rq_ref, rt_ref, k_ref, base_ref, rel_ref):
    ncam = k_ref.shape[0] // 9

    w = bq_ref[0:1, :]
    x = bq_ref[1:2, :]
    y = bq_ref[2:3, :]
    z = bq_ref[3:4, :]
    inv = lax.rsqrt(w * w + x * x + y * y + z * z)
    w, x, y, z = w * inv, x * inv, y * inv, z * inv
    rb = _quat_planes(w, x, y, z)

    wr = rq_ref[:, 0:1]
    xr = rq_ref[:, 1:2]
    yr = rq_ref[:, 2:3]
    zr = rq_ref[:, 3:4]
    invr = lax.rsqrt(wr * wr + xr * xr + yr * yr + zr * zr)
    wr, xr, yr, zr = wr * invr, xr * invr, yr * invr, zr * invr
    rr = _quat_planes(wr, xr, yr, zr)

    rc = [None] * 9
    for i in range(3):
        for l in range(3):
            acc = rr[0 + l] * rb[3 * i + 0]
            acc = acc + rr[3 + l] * rb[3 * i + 1]
            acc = acc + rr[6 + l] * rb[3 * i + 2]
            rc[3 * i + l] = acc

    bt = [bt_ref[i:i + 1, :] for i in range(3)]
    rt = [rt_ref[:, i:i + 1] for i in range(3)]
    tc = [None] * 3
    for i in range(3):
        acc = bt[i] + rc[3 * i + 0] * rt[0]
        acc = acc + rc[3 * i + 1] * rt[1]
        acc = acc + rc[3 * i + 2] * rt[2]
        tc[i] = acc

    for c in range(ncam):
        kk = [k_ref[9 * c + e] for e in range(9)]
        for i in range(3):
            for l in range(3):
                row = c * 12 + 3 * i + l
                mb = kk[3 * i] * rb[l] + kk[3 * i + 1] * rb[3 + l] + kk[3 * i + 2] * rb[6 + l]
                base_ref[row:row + 1, :] = mb
                mr = kk[3 * i] * rc[l] + kk[3 * i + 1] * rc[3 + l] + kk[3 * i + 2] * rc[6 + l]
                rel_ref[row, :, :] = mr
            row = c * 12 + 9 + i
            vb = kk[3 * i] * bt[0] + kk[3 * i + 1] * bt[1] + kk[3 * i + 2] * bt[2]
            base_ref[row:row + 1, :] = vb
            vr = kk[3 * i] * tc[0] + kk[3 * i + 1] * tc[1] + kk[3 * i + 2] * tc[2]
            rel_ref[row, :, :] = vr


def _build_table(base_quat, base_trans, rel_quat, rel_trans, intrinsics):
    nb = base_quat.shape[0]
    nr = rel_quat.shape[0]
    ncam = intrinsics.shape[0]
    base_p, rel_p = pl.pallas_call(
        _table_body,
        out_shape=[
            jax.ShapeDtypeStruct((ncam * 12, nb), jnp.float32),
            jax.ShapeDtypeStruct((ncam * 12, nr, nb), jnp.float32),
        ],
        in_specs=[
            pl.BlockSpec(memory_space=pltpu.VMEM),
            pl.BlockSpec(memory_space=pltpu.VMEM),
            pl.BlockSpec(memory_space=pltpu.VMEM),
            pl.BlockSpec(memory_space=pltpu.VMEM),
            pl.BlockSpec(memory_space=pltpu.SMEM),
        ],
    )(base_quat.T, base_trans.T, rel_quat, rel_trans,
      intrinsics.reshape(ncam * 9))
    tb = base_p.reshape(ncam, 12, nb).transpose(0, 2, 1)
    tr = rel_p.reshape(ncam, 12, nr, nb).transpose(0, 2, 3, 1)
    full = jnp.concatenate([tb, tr.reshape(ncam, nr * nb, 12)], axis=1)
    full = jnp.pad(full, ((0, 0), (0, 0), (0, 4)))
    return full.reshape(ncam * (1 + nr) * nb, 16)


def _pts_pad_body(in_ref, out_ref):
    out_ref[:, 0:3] = in_ref[:]
    out_ref[:, 3:16] = jnp.zeros_like(out_ref[:, 3:16])


def _pad_points(points):
    npts = points.shape[0]
    blk = 4000
    assert npts % blk == 0
    return pl.pallas_call(
        _pts_pad_body,
        out_shape=jax.ShapeDtypeStruct((npts, 16), jnp.float32),
        grid=(npts // blk,),
        in_specs=[pl.BlockSpec((blk, 3), lambda i: (i, 0))],
        out_specs=pl.BlockSpec((blk, 16), lambda i: (i, 0)),
    )(points)


def _make_sc_kernel(n, nb, nr):
    jrows = (1 + nr) * nb
    C = _CHUNK
    n_full = n // (C * _NW)
    w_per = n_full * C
    rem = n - w_per * _NW
    assert rem <= 2 * C and n % 8 == 0
    tail1 = w_per * _NW
    tail2 = n - C

    @functools.partial(
        pl.kernel,
        out_type=jax.ShapeDtypeStruct((2 * n,), jnp.float32),
        mesh=plsc.VectorSubcoreMesh(core_axis_name="c", subcore_axis_name="s",
                                    num_cores=_NC, num_subcores=_NS),
        compiler_params=pltpu.CompilerParams(
            needs_layout_passes=False, use_tc_tiling_on_sc=False),
        scratch_types=[
            pltpu.VMEM((C,), jnp.int32),
            pltpu.VMEM((C,), jnp.int32),
            pltpu.VMEM((C,), jnp.int32),
            pltpu.VMEM((C,), jnp.int32),
            pltpu.VMEM((C,), jnp.int32),
            pltpu.VMEM((C,), jnp.int32),
            pltpu.VMEM((2 * C,), jnp.float32),
            pltpu.VMEM((C, 16), jnp.float32),
            pltpu.VMEM((C, 16), jnp.float32),
            pltpu.VMEM((2 * C,), jnp.float32),
            pltpu.SemaphoreType.DMA,
            pltpu.SemaphoreType.DMA,
        ],
    )
    def sc_kernel(b_hbm, r_hbm, m_hbm, c_hbm, p_hbm, meas_hbm, table_hbm,
                  pts_hbm, out_hbm, b_v, r_v, m_v, c_v, p_v, j_v, meas_v,
                  rows_v, pts_v, out_v, sem_a, sem_b):
        wid = lax.axis_index("s") * _NC + lax.axis_index("c")
        base = wid * w_per
        iota = lax.iota(jnp.int32, _L)

        def do_chunk(off):
            pltpu.sync_copy(b_hbm.at[pl.ds(off, C)], b_v)
            pltpu.sync_copy(r_hbm.at[pl.ds(off, C)], r_v)
            pltpu.sync_copy(m_hbm.at[pl.ds(off, C)], m_v)
            pltpu.sync_copy(c_hbm.at[pl.ds(off, C)], c_v)
            pltpu.sync_copy(p_hbm.at[pl.ds(off, C)], p_v)
            pltpu.sync_copy(meas_hbm.at[pl.ds(2 * off, 2 * C)], meas_v)

            def j_body(v, _):
                o = v * _L
                bb = b_v[pl.ds(o, _L)]
                rr = r_v[pl.ds(o, _L)]
                mm = m_v[pl.ds(o, _L)]
                cc = c_v[pl.ds(o, _L)]
                j = jnp.where(mm > 0, (rr + 1) * nb + bb, bb)
                j_v[pl.ds(o, _L)] = j + cc * jrows
                return 0

            lax.fori_loop(0, C // _L, j_body, 0)

            cp_rows = pltpu.make_async_copy(table_hbm.at[j_v], rows_v, sem_a)
            cp_rows.start()
            cp_pts = pltpu.make_async_copy(pts_hbm.at[p_v], pts_v, sem_b)
            cp_pts.start()
            cp_rows.wait()
            cp_pts.wait()

            def c_body(v, _):
                o = v * _L
                row = o + iota
                m = [plsc.load_gather(rows_v, [row, jnp.full((_L,), k, jnp.int32)])
                     for k in range(12)]
                px = plsc.load_gather(pts_v, [row, jnp.full((_L,), 0, jnp.int32)])
                py = plsc.load_gather(pts_v, [row, jnp.full((_L,), 1, jnp.int32)])
                pz = plsc.load_gather(pts_v, [row, jnp.full((_L,), 2, jnp.int32)])
                xn = m[0] * px + m[1] * py + m[2] * pz + m[9]
                yn = m[3] * px + m[4] * py + m[5] * pz + m[10]
                zn = m[6] * px + m[7] * py + m[8] * pz + m[11]
                inv = 1.0 / zn
                mx = plsc.load_gather(meas_v, [2 * row])
                my = plsc.load_gather(meas_v, [2 * row + 1])
                plsc.store_scatter(out_v, [2 * row], xn * inv - mx)
                plsc.store_scatter(out_v, [2 * row + 1], yn * inv - my)
                return 0

            lax.fori_loop(0, C // _L, c_body, 0)
            pltpu.sync_copy(out_v, out_hbm.at[pl.ds(2 * off, 2 * C)])

        def chunk_body(g, _):
            do_chunk(base + g * C)
            return 0

        lax.fori_loop(0, n_full, chunk_body, 0)
        if rem > 0:
            @pl.when(wid == 0)
            def _():
                do_chunk(jnp.int32(tail1))

            @pl.when(wid == 1)
            def _():
                do_chunk(jnp.int32(tail2))

    return sc_kernel


def kernel(cam_indices, pt_indices, base_pose_indices, relative_pose_indices,
           is_relative, pixels_measured, base_quat, base_trans, rel_quat,
           rel_trans, points, intrinsics):
    n = cam_indices.shape[0]
    nb = base_quat.shape[0]
    nr = rel_quat.shape[0]

    table = _build_table(base_quat, base_trans, rel_quat, rel_trans, intrinsics)
    pts_pad = _pad_points(points)

    b_i = base_pose_indices.astype(jnp.int32)
    r_i = relative_pose_indices.astype(jnp.int32)
    m_i = is_relative.astype(jnp.int32)
    c_i = cam_indices.astype(jnp.int32)
    p_i = pt_indices.astype(jnp.int32)
    meas = pixels_measured.reshape(-1)

    sc = _make_sc_kernel(n, nb, nr)
    return sc(b_i, r_i, m_i, c_i, p_i, meas, table, pts_pad)


# speedup vs baseline: 15.3665x; 1.2269x over previous
import functools

import jax
import jax.numpy as jnp
from jax import lax
from jax.experimental import pallas as pl
from jax.experimental.pallas import tpu as pltpu
from jax.experimental.pallas import tpu_sc as plsc

_NC = 2
_NS = 16
_NW = _NC * _NS
_L = 16
_CHUNK = 512


def _quat_planes(w, x, y, z):
    return [
        1.0 - 2.0 * (y * y + z * z), 2.0 * (x * y - w * z), 2.0 * (x * z + w * y),
        2.0 * (x * y + w * z), 1.0 - 2.0 * (x * x + z * z), 2.0 * (y * z - w * x),
        2.0 * (x * z - w * y), 2.0 * (y * z + w * x), 1.0 - 2.0 * (x * x + y * y),
    ]


def _table_body(bq_ref, bt_ref, rq_ref, rt_ref, k_ref, base_ref, rel_ref):
    ncam = k_ref.shape[0] // 9

    w = bq_ref[0:1, :]
    x = bq_ref[1:2, :]
    y = bq_ref[2:3, :]
    z = bq_ref[3:4, :]
    inv = lax.rsqrt(w * w + x * x + y * y + z * z)
    w, x, y, z = w * inv, x * inv, y * inv, z * inv
    rb = _quat_planes(w, x, y, z)

    wr = rq_ref[:, 0:1]
    xr = rq_ref[:, 1:2]
    yr = rq_ref[:, 2:3]
    zr = rq_ref[:, 3:4]
    invr = lax.rsqrt(wr * wr + xr * xr + yr * yr + zr * zr)
    wr, xr, yr, zr = wr * invr, xr * invr, yr * invr, zr * invr
    rr = _quat_planes(wr, xr, yr, zr)

    rc = [None] * 9
    for i in range(3):
        for l in range(3):
            acc = rr[0 + l] * rb[3 * i + 0]
            acc = acc + rr[3 + l] * rb[3 * i + 1]
            acc = acc + rr[6 + l] * rb[3 * i + 2]
            rc[3 * i + l] = acc

    bt = [bt_ref[i:i + 1, :] for i in range(3)]
    rt = [rt_ref[:, i:i + 1] for i in range(3)]
    tc = [None] * 3
    for i in range(3):
        acc = bt[i] + rc[3 * i + 0] * rt[0]
        acc = acc + rc[3 * i + 1] * rt[1]
        acc = acc + rc[3 * i + 2] * rt[2]
        tc[i] = acc

    for c in range(ncam):
        kk = [k_ref[9 * c + e] for e in range(9)]
        for i in range(3):
            for l in range(3):
                row = c * 12 + 3 * i + l
                mb = kk[3 * i] * rb[l] + kk[3 * i + 1] * rb[3 + l] + kk[3 * i + 2] * rb[6 + l]
                base_ref[row:row + 1, :] = mb
                mr = kk[3 * i] * rc[l] + kk[3 * i + 1] * rc[3 + l] + kk[3 * i + 2] * rc[6 + l]
                rel_ref[row, :, :] = mr
            row = c * 12 + 9 + i
            vb = kk[3 * i] * bt[0] + kk[3 * i + 1] * bt[1] + kk[3 * i + 2] * bt[2]
            base_ref[row:row + 1, :] = vb
            vr = kk[3 * i] * tc[0] + kk[3 * i + 1] * tc[1] + kk[3 * i + 2] * tc[2]
            rel_ref[row, :, :] = vr


def _build_table(base_quat, base_trans, rel_quat, rel_trans, intrinsics):
    nb = base_quat.shape[0]
    nr = rel_quat.shape[0]
    ncam = intrinsics.shape[0]
    base_p, rel_p = pl.pallas_call(
        _table_body,
        out_shape=[
            jax.ShapeDtypeStruct((ncam * 12, nb), jnp.float32),
            jax.ShapeDtypeStruct((ncam * 12, nr, nb), jnp.float32),
        ],
        in_specs=[
            pl.BlockSpec(memory_space=pltpu.VMEM),
            pl.BlockSpec(memory_space=pltpu.VMEM),
            pl.BlockSpec(memory_space=pltpu.VMEM),
            pl.BlockSpec(memory_space=pltpu.VMEM),
            pl.BlockSpec(memory_space=pltpu.SMEM),
        ],
    )(base_quat.T, base_trans.T, rel_quat, rel_trans,
      intrinsics.reshape(ncam * 9))
    tb = base_p.reshape(ncam, 12, nb).transpose(0, 2, 1)
    tr = rel_p.reshape(ncam, 12, nr, nb).transpose(0, 2, 3, 1)
    full = jnp.concatenate([tb, tr.reshape(ncam, nr * nb, 12)], axis=1)
    full = jnp.pad(full, ((0, 0), (0, 0), (0, 4)))
    return full.reshape(ncam * (1 + nr) * nb, 16)


def _pts_pad_body(in_ref, out_ref):
    out_ref[:, 0:3] = in_ref[:]
    out_ref[:, 3:16] = jnp.zeros_like(out_ref[:, 3:16])


def _pad_points(points):
    npts = points.shape[0]
    blk = 4000
    assert npts % blk == 0
    return pl.pallas_call(
        _pts_pad_body,
        out_shape=jax.ShapeDtypeStruct((npts, 16), jnp.float32),
        grid=(npts // blk,),
        in_specs=[pl.BlockSpec((blk, 3), lambda i: (i, 0))],
        out_specs=pl.BlockSpec((blk, 16), lambda i: (i, 0)),
    )(points)


def _make_sc_kernel(n, nb, nr):
    jrows = (1 + nr) * nb
    C = _CHUNK
    n_full = n // (C * _NW)
    w_per = n_full * C
    rem = n - w_per * _NW
    assert rem <= 2 * C and n % 8 == 0
    tail1 = w_per * _NW
    tail2 = n - C

    @functools.partial(
        pl.kernel,
        out_type=jax.ShapeDtypeStruct((2 * n,), jnp.float32),
        mesh=plsc.VectorSubcoreMesh(core_axis_name="c", subcore_axis_name="s",
                                    num_cores=_NC, num_subcores=_NS),
        compiler_params=pltpu.CompilerParams(
            needs_layout_passes=False, use_tc_tiling_on_sc=False),
        scratch_types=[
            pltpu.VMEM((C,), jnp.int32),
            pltpu.VMEM((C,), jnp.int32),
            pltpu.VMEM((C,), jnp.int32),
            pltpu.VMEM((C,), jnp.int32),
            pltpu.VMEM((C,), jnp.int32),
            pltpu.VMEM((C,), jnp.int32),
            pltpu.VMEM((C, 16), jnp.float32),
            pltpu.VMEM((C, 16), jnp.float32),
            pltpu.VMEM((2 * C,), jnp.float32),
            pltpu.SemaphoreType.DMA,
            pltpu.SemaphoreType.DMA,
        ],
    )
    def sc_kernel(b_hbm, r_hbm, m_hbm, c_hbm, p_hbm, table_hbm,
                  pts_hbm, out_hbm, b_v, r_v, m_v, c_v, p_v, j_v,
                  rows_v, pts_v, out_v, sem_a, sem_b):
        wid = lax.axis_index("s") * _NC + lax.axis_index("c")
        base = wid * w_per
        iota = lax.iota(jnp.int32, _L)

        def do_chunk(off):
            pltpu.sync_copy(b_hbm.at[pl.ds(off, C)], b_v)
            pltpu.sync_copy(r_hbm.at[pl.ds(off, C)], r_v)
            pltpu.sync_copy(m_hbm.at[pl.ds(off, C)], m_v)
            pltpu.sync_copy(c_hbm.at[pl.ds(off, C)], c_v)
            pltpu.sync_copy(p_hbm.at[pl.ds(off, C)], p_v)

            def j_body(v, _):
                o = v * _L
                bb = b_v[pl.ds(o, _L)]
                rr = r_v[pl.ds(o, _L)]
                mm = m_v[pl.ds(o, _L)]
                cc = c_v[pl.ds(o, _L)]
                j = jnp.where(mm > 0, (rr + 1) * nb + bb, bb)
                j_v[pl.ds(o, _L)] = j + cc * jrows
                return 0

            lax.fori_loop(0, C // _L, j_body, 0)

            cp_rows = pltpu.make_async_copy(table_hbm.at[j_v], rows_v, sem_a)
            cp_rows.start()
            cp_pts = pltpu.make_async_copy(pts_hbm.at[p_v], pts_v, sem_b)
            cp_pts.start()
            cp_rows.wait()
            cp_pts.wait()

            def c_body(v, _):
                o = v * _L
                row = o + iota
                m = [plsc.load_gather(rows_v, [row, jnp.full((_L,), k, jnp.int32)])
                     for k in range(12)]
                px = plsc.load_gather(pts_v, [row, jnp.full((_L,), 0, jnp.int32)])
                py = plsc.load_gather(pts_v, [row, jnp.full((_L,), 1, jnp.int32)])
                pz = plsc.load_gather(pts_v, [row, jnp.full((_L,), 2, jnp.int32)])
                xn = m[0] * px + m[1] * py + m[2] * pz + m[9]
                yn = m[3] * px + m[4] * py + m[5] * pz + m[10]
                zn = m[6] * px + m[7] * py + m[8] * pz + m[11]
                inv = 1.0 / zn
                plsc.store_scatter(out_v, [2 * row], xn * inv)
                plsc.store_scatter(out_v, [2 * row + 1], yn * inv)
                return 0

            lax.fori_loop(0, C // _L, c_body, 0)
            pltpu.sync_copy(out_v, out_hbm.at[pl.ds(2 * off, 2 * C)])

        def chunk_body(g, _):
            do_chunk(base + g * C)
            return 0

        lax.fori_loop(0, n_full, chunk_body, 0)
        if rem > 0:
            @pl.when(wid == 0)
            def _():
                do_chunk(jnp.int32(tail1))

            @pl.when(wid == 1)
            def _():
                do_chunk(jnp.int32(tail2))

    return sc_kernel


def kernel(cam_indices, pt_indices, base_pose_indices, relative_pose_indices,
           is_relative, pixels_measured, base_quat, base_trans, rel_quat,
           rel_trans, points, intrinsics):
    n = cam_indices.shape[0]
    nb = base_quat.shape[0]
    nr = rel_quat.shape[0]

    table = _build_table(base_quat, base_trans, rel_quat, rel_trans, intrinsics)
    pts_pad = _pad_points(points)

    b_i = base_pose_indices.astype(jnp.int32)
    r_i = relative_pose_indices.astype(jnp.int32)
    m_i = is_relative.astype(jnp.int32)
    c_i = cam_indices.astype(jnp.int32)
    p_i = pt_indices.astype(jnp.int32)

    sc = _make_sc_kernel(n, nb, nr)
    pred = sc(b_i, r_i, m_i, c_i, p_i, table, pts_pad)
    return pred - pixels_measured.reshape(-1)


# speedup vs baseline: 15.5791x; 1.0138x over previous
import functools

import jax
import jax.numpy as jnp
from jax import lax
from jax.experimental import pallas as pl
from jax.experimental.pallas import tpu as pltpu
from jax.experimental.pallas import tpu_sc as plsc

_NC = 2
_NS = 16
_NW = _NC * _NS
_L = 16
_CHUNK = 512
_PCHUNK = 800

_SC_PARAMS = pltpu.CompilerParams(
    needs_layout_passes=False, use_tc_tiling_on_sc=False)


def _quat_planes(w, x, y, z):
    return [
        1.0 - 2.0 * (y * y + z * z), 2.0 * (x * y - w * z), 2.0 * (x * z + w * y),
        2.0 * (x * y + w * z), 1.0 - 2.0 * (x * x + z * z), 2.0 * (y * z - w * x),
        2.0 * (x * z - w * y), 2.0 * (y * z + w * x), 1.0 - 2.0 * (x * x + y * y),
    ]


def _planes_body(bq_ref, bt_ref, rq_ref, rt_ref, k_ref, pts_ref,
                 base_ref, rel_ref, pts1_ref):
    ncam = k_ref.shape[0] // 9
    nb = bq_ref.shape[1]
    nr = rq_ref.shape[0]
    npts = pts_ref.shape[1]

    for i in range(3):
        pts1_ref[pl.ds(i * npts, npts)] = pts_ref[i:i + 1, :].reshape(npts)

    w = bq_ref[0:1, :]
    x = bq_ref[1:2, :]
    y = bq_ref[2:3, :]
    z = bq_ref[3:4, :]
    inv = lax.rsqrt(w * w + x * x + y * y + z * z)
    w, x, y, z = w * inv, x * inv, y * inv, z * inv
    rb = _quat_planes(w, x, y, z)

    wr = rq_ref[:, 0:1]
    xr = rq_ref[:, 1:2]
    yr = rq_ref[:, 2:3]
    zr = rq_ref[:, 3:4]
    invr = lax.rsqrt(wr * wr + xr * xr + yr * yr + zr * zr)
    wr, xr, yr, zr = wr * invr, xr * invr, yr * invr, zr * invr
    rr = _quat_planes(wr, xr, yr, zr)

    rc = [None] * 9
    for i in range(3):
        for l in range(3):
            acc = rr[0 + l] * rb[3 * i + 0]
            acc = acc + rr[3 + l] * rb[3 * i + 1]
            acc = acc + rr[6 + l] * rb[3 * i + 2]
            rc[3 * i + l] = acc

    bt = [bt_ref[i:i + 1, :] for i in range(3)]
    rt = [rt_ref[:, i:i + 1] for i in range(3)]
    tc = [None] * 3
    for i in range(3):
        acc = bt[i] + rc[3 * i + 0] * rt[0]
        acc = acc + rc[3 * i + 1] * rt[1]
        acc = acc + rc[3 * i + 2] * rt[2]
        tc[i] = acc

    def wr_base(row, val):
        base_ref[pl.ds(row * nb, nb)] = val.reshape(nb)

    def wr_rel(row, val):
        for r in range(nr):
            rel_ref[pl.ds((row * nr + r) * nb, nb)] = val[r:r + 1, :].reshape(nb)

    for c in range(ncam):
        kk = [k_ref[9 * c + e] for e in range(9)]
        for i in range(3):
            for l in range(3):
                row = c * 12 + 3 * i + l
                wr_base(row, kk[3 * i] * rb[l] + kk[3 * i + 1] * rb[3 + l]
                        + kk[3 * i + 2] * rb[6 + l])
                wr_rel(row, kk[3 * i] * rc[l] + kk[3 * i + 1] * rc[3 + l]
                       + kk[3 * i + 2] * rc[6 + l])
            row = c * 12 + 9 + i
            wr_base(row, kk[3 * i] * bt[0] + kk[3 * i + 1] * bt[1]
                    + kk[3 * i + 2] * bt[2])
            wr_rel(row, kk[3 * i] * tc[0] + kk[3 * i + 1] * tc[1]
                   + kk[3 * i + 2] * tc[2])


def _build_planes(base_quat, base_trans, rel_quat, rel_trans, intrinsics,
                  points):
    nb = base_quat.shape[0]
    nr = rel_quat.shape[0]
    ncam = intrinsics.shape[0]
    npts = points.shape[0]
    return pl.pallas_call(
        _planes_body,
        out_shape=[
            jax.ShapeDtypeStruct((ncam * 12 * nb,), jnp.float32),
            jax.ShapeDtypeStruct((ncam * 12 * nr * nb,), jnp.float32),
            jax.ShapeDtypeStruct((3 * npts,), jnp.float32),
        ],
        in_specs=[
            pl.BlockSpec(memory_space=pltpu.VMEM),
            pl.BlockSpec(memory_space=pltpu.VMEM),
            pl.BlockSpec(memory_space=pltpu.VMEM),
            pl.BlockSpec(memory_space=pltpu.VMEM),
            pl.BlockSpec(memory_space=pltpu.SMEM),
            pl.BlockSpec(memory_space=pltpu.VMEM),
        ],
    )(base_quat.T, base_trans.T, rel_quat, rel_trans,
      intrinsics.reshape(ncam * 9), points.T)


def _make_prep_kernel(nb, nr, ncam, npts):
    jrows = (1 + nr) * nb
    n_psub = npts // _PCHUNK
    psub_per_w = -(-n_psub // _NW)
    assert npts % _PCHUNK == 0

    @functools.partial(
        pl.kernel,
        out_type=[
            jax.ShapeDtypeStruct((ncam * jrows, 16), jnp.float32),
            jax.ShapeDtypeStruct((npts, 16), jnp.float32),
        ],
        mesh=plsc.VectorSubcoreMesh(core_axis_name="c", subcore_axis_name="s",
                                    num_cores=_NC, num_subcores=_NS),
        compiler_params=_SC_PARAMS,
        scratch_types=[
            pltpu.VMEM((12, nb), jnp.float32),
            pltpu.VMEM((nb, 16), jnp.float32),
            pltpu.VMEM((3, _PCHUNK), jnp.float32),
            pltpu.VMEM((_PCHUNK, 16), jnp.float32),
        ],
    )
    def prep(base1_hbm, rel1_hbm, pts1_hbm, table_hbm, pts16_hbm,
             plane_v, seg_v, ppl_v, pbuf_v):
        wid = lax.axis_index("s") * _NC + lax.axis_index("c")
        iota = lax.iota(jnp.int32, _L)

        def interleave(src_v, dst_v, nrows, nent):
            def body(v, _):
                o = v * _L
                row = o + iota
                for e in range(nent):
                    plsc.store_scatter(
                        dst_v, [row, jnp.full((_L,), e, jnp.int32)],
                        src_v[e, pl.ds(o, _L)])
                return 0
            lax.fori_loop(0, nrows // _L, body, 0)

        c_rel = wid // nr
        r_rel = wid % nr
        for e in range(12):
            src = ((c_rel * 12 + e) * nr + r_rel) * nb
            pltpu.sync_copy(rel1_hbm.at[pl.ds(src, nb)], plane_v.at[e])
        interleave(plane_v, seg_v, nb, 12)
        dst_row = c_rel * jrows + (1 + r_rel) * nb
        pltpu.sync_copy(seg_v, table_hbm.at[pl.ds(dst_row, nb)])

        @pl.when(wid < ncam)
        def _():
            for e in range(12):
                src = (wid * 12 + e) * nb
                pltpu.sync_copy(base1_hbm.at[pl.ds(src, nb)], plane_v.at[e])
            interleave(plane_v, seg_v, nb, 12)
            pltpu.sync_copy(seg_v, table_hbm.at[pl.ds(wid * jrows, nb)])

        for s in range(psub_per_w):
            sub = wid * psub_per_w + s

            @pl.when(sub < n_psub)
            def _():
                row0 = sub * _PCHUNK
                for i in range(3):
                    pltpu.sync_copy(
                        pts1_hbm.at[pl.ds(i * npts + row0, _PCHUNK)],
                        ppl_v.at[i])
                interleave(ppl_v, pbuf_v, _PCHUNK, 3)
                pltpu.sync_copy(pbuf_v, pts16_hbm.at[pl.ds(row0, _PCHUNK)])

    return prep


def _make_sc_kernel(n, nb, nr):
    jrows = (1 + nr) * nb
    C = _CHUNK
    n_full = n // (C * _NW)
    w_per = n_full * C
    rem = n - w_per * _NW
    assert rem <= 2 * C and n % 8 == 0
    tail1 = w_per * _NW
    tail2 = n - C

    @functools.partial(
        pl.kernel,
        out_type=jax.ShapeDtypeStruct((2 * n,), jnp.float32),
        mesh=plsc.VectorSubcoreMesh(core_axis_name="c", subcore_axis_name="s",
                                    num_cores=_NC, num_subcores=_NS),
        compiler_params=_SC_PARAMS,
        scratch_types=[
            pltpu.VMEM((C,), jnp.int32),
            pltpu.VMEM((C,), jnp.int32),
            pltpu.VMEM((C,), jnp.int32),
            pltpu.VMEM((C,), jnp.int32),
            pltpu.VMEM((C,), jnp.int32),
            pltpu.VMEM((C,), jnp.int32),
            pltpu.VMEM((C, 16), jnp.float32),
            pltpu.VMEM((C, 16), jnp.float32),
            pltpu.VMEM((2 * C,), jnp.float32),
            pltpu.SemaphoreType.DMA,
            pltpu.SemaphoreType.DMA,
        ],
    )
    def sc_kernel(b_hbm, r_hbm, m_hbm, c_hbm, p_hbm, table_hbm,
                  pts_hbm, out_hbm, b_v, r_v, m_v, c_v, p_v, j_v,
                  rows_v, pts_v, out_v, sem_a, sem_b):
        wid = lax.axis_index("s") * _NC + lax.axis_index("c")
        base = wid * w_per
        iota = lax.iota(jnp.int32, _L)

        def do_chunk(off):
            pltpu.sync_copy(b_hbm.at[pl.ds(off, C)], b_v)
            pltpu.sync_copy(r_hbm.at[pl.ds(off, C)], r_v)
            pltpu.sync_copy(m_hbm.at[pl.ds(off, C)], m_v)
            pltpu.sync_copy(c_hbm.at[pl.ds(off, C)], c_v)
            pltpu.sync_copy(p_hbm.at[pl.ds(off, C)], p_v)

            def j_body(v, _):
                o = v * _L
                bb = b_v[pl.ds(o, _L)]
                rr = r_v[pl.ds(o, _L)]
                mm = m_v[pl.ds(o, _L)]
                cc = c_v[pl.ds(o, _L)]
                j = jnp.where(mm > 0, (rr + 1) * nb + bb, bb)
                j_v[pl.ds(o, _L)] = j + cc * jrows
                return 0

            lax.fori_loop(0, C // _L, j_body, 0)

            cp_rows = pltpu.make_async_copy(table_hbm.at[j_v], rows_v, sem_a)
            cp_rows.start()
            cp_pts = pltpu.make_async_copy(pts_hbm.at[p_v], pts_v, sem_b)
            cp_pts.start()
            cp_rows.wait()
            cp_pts.wait()

            def c_body(v, _):
                o = v * _L
                row = o + iota
                m = [plsc.load_gather(rows_v, [row, jnp.full((_L,), k, jnp.int32)])
                     for k in range(12)]
                px = plsc.load_gather(pts_v, [row, jnp.full((_L,), 0, jnp.int32)])
                py = plsc.load_gather(pts_v, [row, jnp.full((_L,), 1, jnp.int32)])
                pz = plsc.load_gather(pts_v, [row, jnp.full((_L,), 2, jnp.int32)])
                xn = m[0] * px + m[1] * py + m[2] * pz + m[9]
                yn = m[3] * px + m[4] * py + m[5] * pz + m[10]
                zn = m[6] * px + m[7] * py + m[8] * pz + m[11]
                inv = 1.0 / zn
                plsc.store_scatter(out_v, [2 * row], xn * inv)
                plsc.store_scatter(out_v, [2 * row + 1], yn * inv)
                return 0

            lax.fori_loop(0, C // _L, c_body, 0)
            pltpu.sync_copy(out_v, out_hbm.at[pl.ds(2 * off, 2 * C)])

        def chunk_body(g, _):
            do_chunk(base + g * C)
            return 0

        lax.fori_loop(0, n_full, chunk_body, 0)
        if rem > 0:
            @pl.when(wid == 0)
            def _():
                do_chunk(jnp.int32(tail1))

            @pl.when(wid == 1)
            def _():
                do_chunk(jnp.int32(tail2))

    return sc_kernel


def kernel(cam_indices, pt_indices, base_pose_indices, relative_pose_indices,
           is_relative, pixels_measured, base_quat, base_trans, rel_quat,
           rel_trans, points, intrinsics):
    n = cam_indices.shape[0]
    nb = base_quat.shape[0]
    nr = rel_quat.shape[0]
    ncam = intrinsics.shape[0]
    npts = points.shape[0]
    assert _NW == nr * ncam

    base1, rel1, pts1 = _build_planes(base_quat, base_trans, rel_quat,
                                      rel_trans, intrinsics, points)
    table, pts16 = _make_prep_kernel(nb, nr, ncam, npts)(base1, rel1, pts1)

    b_i = base_pose_indices.astype(jnp.int32)
    r_i = relative_pose_indices.astype(jnp.int32)
    m_i = is_relative.astype(jnp.int32)
    c_i = cam_indices.astype(jnp.int32)
    p_i = pt_indices.astype(jnp.int32)

    sc = _make_sc_kernel(n, nb, nr)
    pred = sc(b_i, r_i, m_i, c_i, p_i, table, pts16)
    return pred - pixels_measured.reshape(-1)


# speedup vs baseline: 46.4584x; 2.9821x over previous
import functools

import jax
import jax.numpy as jnp
from jax import lax
from jax.experimental import pallas as pl
from jax.experimental.pallas import tpu as pltpu
from jax.experimental.pallas import tpu_sc as plsc

_NC = 2
_NS = 16
_NW = _NC * _NS
_L = 16
_CHUNK = 512
_PCHUNK = 800

_SC_PARAMS = pltpu.CompilerParams(
    needs_layout_passes=False, use_tc_tiling_on_sc=False)


def _quat_planes(w, x, y, z):
    return [
        1.0 - 2.0 * (y * y + z * z), 2.0 * (x * y - w * z), 2.0 * (x * z + w * y),
        2.0 * (x * y + w * z), 1.0 - 2.0 * (x * x + z * z), 2.0 * (y * z - w * x),
        2.0 * (x * z - w * y), 2.0 * (y * z + w * x), 1.0 - 2.0 * (x * x + y * y),
    ]


def _planes_body(bq_ref, bt_ref, rq_ref, rt_ref, k_ref, pts_ref,
                 base_ref, rel_ref, pts1_ref):
    ncam = k_ref.shape[0] // 9
    nb = bq_ref.shape[1]
    nr = rq_ref.shape[0]
    npts = pts_ref.shape[1]

    for i in range(3):
        pts1_ref[pl.ds(i * npts, npts)] = pts_ref[i:i + 1, :].reshape(npts)

    w = bq_ref[0:1, :]
    x = bq_ref[1:2, :]
    y = bq_ref[2:3, :]
    z = bq_ref[3:4, :]
    inv = lax.rsqrt(w * w + x * x + y * y + z * z)
    w, x, y, z = w * inv, x * inv, y * inv, z * inv
    rb = _quat_planes(w, x, y, z)

    wr = rq_ref[:, 0:1]
    xr = rq_ref[:, 1:2]
    yr = rq_ref[:, 2:3]
    zr = rq_ref[:, 3:4]
    invr = lax.rsqrt(wr * wr + xr * xr + yr * yr + zr * zr)
    wr, xr, yr, zr = wr * invr, xr * invr, yr * invr, zr * invr
    rr = _quat_planes(wr, xr, yr, zr)

    rc = [None] * 9
    for i in range(3):
        for l in range(3):
            acc = rr[0 + l] * rb[3 * i + 0]
            acc = acc + rr[3 + l] * rb[3 * i + 1]
            acc = acc + rr[6 + l] * rb[3 * i + 2]
            rc[3 * i + l] = acc

    bt = [bt_ref[i:i + 1, :] for i in range(3)]
    rt = [rt_ref[:, i:i + 1] for i in range(3)]
    tc = [None] * 3
    for i in range(3):
        acc = bt[i] + rc[3 * i + 0] * rt[0]
        acc = acc + rc[3 * i + 1] * rt[1]
        acc = acc + rc[3 * i + 2] * rt[2]
        tc[i] = acc

    def wr_base(row, val):
        base_ref[pl.ds(row * nb, nb)] = val.reshape(nb)

    def wr_rel(row, val):
        for r in range(nr):
            rel_ref[pl.ds((row * nr + r) * nb, nb)] = val[r:r + 1, :].reshape(nb)

    for c in range(ncam):
        kk = [k_ref[9 * c + e] for e in range(9)]
        for i in range(3):
            for l in range(3):
                row = c * 12 + 3 * i + l
                wr_base(row, kk[3 * i] * rb[l] + kk[3 * i + 1] * rb[3 + l]
                        + kk[3 * i + 2] * rb[6 + l])
                wr_rel(row, kk[3 * i] * rc[l] + kk[3 * i + 1] * rc[3 + l]
                       + kk[3 * i + 2] * rc[6 + l])
            row = c * 12 + 9 + i
            wr_base(row, kk[3 * i] * bt[0] + kk[3 * i + 1] * bt[1]
                    + kk[3 * i + 2] * bt[2])
            wr_rel(row, kk[3 * i] * tc[0] + kk[3 * i + 1] * tc[1]
                   + kk[3 * i + 2] * tc[2])


def _build_planes(base_quat, base_trans, rel_quat, rel_trans, intrinsics,
                  points):
    nb = base_quat.shape[0]
    nr = rel_quat.shape[0]
    ncam = intrinsics.shape[0]
    npts = points.shape[0]
    return pl.pallas_call(
        _planes_body,
        out_shape=[
            jax.ShapeDtypeStruct((ncam * 12 * nb,), jnp.float32),
            jax.ShapeDtypeStruct((ncam * 12 * nr * nb,), jnp.float32),
            jax.ShapeDtypeStruct((3 * npts,), jnp.float32),
        ],
        in_specs=[
            pl.BlockSpec(memory_space=pltpu.VMEM),
            pl.BlockSpec(memory_space=pltpu.VMEM),
            pl.BlockSpec(memory_space=pltpu.VMEM),
            pl.BlockSpec(memory_space=pltpu.VMEM),
            pl.BlockSpec(memory_space=pltpu.SMEM),
            pl.BlockSpec(memory_space=pltpu.VMEM),
        ],
    )(base_quat.T, base_trans.T, rel_quat, rel_trans,
      intrinsics.reshape(ncam * 9), points.T)


def _make_prep_kernel(nb, nr, ncam, npts):
    jrows = (1 + nr) * nb
    n_psub = npts // _PCHUNK
    psub_per_w = -(-n_psub // _NW)
    assert npts % _PCHUNK == 0

    @functools.partial(
        pl.kernel,
        out_type=[
            jax.ShapeDtypeStruct((ncam * jrows, 16), jnp.float32),
            jax.ShapeDtypeStruct((npts, 16), jnp.float32),
        ],
        mesh=plsc.VectorSubcoreMesh(core_axis_name="c", subcore_axis_name="s",
                                    num_cores=_NC, num_subcores=_NS),
        compiler_params=_SC_PARAMS,
        scratch_types=[
            pltpu.VMEM((12, nb), jnp.float32),
            pltpu.VMEM((nb, 16), jnp.float32),
            pltpu.VMEM((3, _PCHUNK), jnp.float32),
            pltpu.VMEM((_PCHUNK, 16), jnp.float32),
        ],
    )
    def prep(base1_hbm, rel1_hbm, pts1_hbm, table_hbm, pts16_hbm,
             plane_v, seg_v, ppl_v, pbuf_v):
        wid = lax.axis_index("s") * _NC + lax.axis_index("c")
        iota = lax.iota(jnp.int32, _L)

        def interleave(src_v, dst_v, nrows, nent):
            def body(v, _):
                o = v * _L
                row = o + iota
                for e in range(nent):
                    plsc.store_scatter(
                        dst_v, [row, jnp.full((_L,), e, jnp.int32)],
                        src_v[e, pl.ds(o, _L)])
                return 0
            lax.fori_loop(0, nrows // _L, body, 0)

        c_rel = wid // nr
        r_rel = wid % nr
        for e in range(12):
            src = ((c_rel * 12 + e) * nr + r_rel) * nb
            pltpu.sync_copy(rel1_hbm.at[pl.ds(src, nb)], plane_v.at[e])
        interleave(plane_v, seg_v, nb, 12)
        dst_row = c_rel * jrows + (1 + r_rel) * nb
        pltpu.sync_copy(seg_v, table_hbm.at[pl.ds(dst_row, nb)])

        @pl.when(wid < ncam)
        def _():
            for e in range(12):
                src = (wid * 12 + e) * nb
                pltpu.sync_copy(base1_hbm.at[pl.ds(src, nb)], plane_v.at[e])
            interleave(plane_v, seg_v, nb, 12)
            pltpu.sync_copy(seg_v, table_hbm.at[pl.ds(wid * jrows, nb)])

        for s in range(psub_per_w):
            sub = wid * psub_per_w + s

            @pl.when(sub < n_psub)
            def _():
                row0 = sub * _PCHUNK
                for i in range(3):
                    pltpu.sync_copy(
                        pts1_hbm.at[pl.ds(i * npts + row0, _PCHUNK)],
                        ppl_v.at[i])
                interleave(ppl_v, pbuf_v, _PCHUNK, 3)
                pltpu.sync_copy(pbuf_v, pts16_hbm.at[pl.ds(row0, _PCHUNK)])

    return prep


def _make_sc_kernel(n, nb, nr):
    jrows = (1 + nr) * nb
    C = _CHUNK
    n_full = n // (C * _NW)
    w_per = n_full * C
    rem = n - w_per * _NW
    assert rem <= 2 * C and n % 8 == 0
    tail1 = w_per * _NW
    tail2 = n - C

    @functools.partial(
        pl.kernel,
        out_type=jax.ShapeDtypeStruct((2 * n,), jnp.float32),
        mesh=plsc.VectorSubcoreMesh(core_axis_name="c", subcore_axis_name="s",
                                    num_cores=_NC, num_subcores=_NS),
        compiler_params=_SC_PARAMS,
        scratch_types=[
            pltpu.VMEM((C,), jnp.int32),
            pltpu.VMEM((C,), jnp.int32),
            pltpu.VMEM((C,), jnp.int32),
            pltpu.VMEM((C,), jnp.int32),
            pltpu.VMEM((C,), jnp.int32),
            pltpu.VMEM((C,), jnp.int32),
            pltpu.VMEM((C,), jnp.float32),
            pltpu.VMEM((C,), jnp.float32),
            pltpu.VMEM((C, 16), jnp.float32),
            pltpu.VMEM((C, 16), jnp.float32),
            pltpu.VMEM((2 * C,), jnp.float32),
            pltpu.SemaphoreType.DMA,
            pltpu.SemaphoreType.DMA,
        ],
    )
    def sc_kernel(b_hbm, r_hbm, m_hbm, c_hbm, p_hbm, mx_hbm, my_hbm,
                  table_hbm, pts_hbm, out_hbm, b_v, r_v, m_v, c_v, p_v, j_v,
                  mx_v, my_v, rows_v, pts_v, out_v, sem_a, sem_b):
        wid = lax.axis_index("s") * _NC + lax.axis_index("c")
        base = wid * w_per
        iota = lax.iota(jnp.int32, _L)

        def do_chunk(off):
            pltpu.sync_copy(b_hbm.at[pl.ds(off, C)], b_v)
            pltpu.sync_copy(r_hbm.at[pl.ds(off, C)], r_v)
            pltpu.sync_copy(m_hbm.at[pl.ds(off, C)], m_v)
            pltpu.sync_copy(c_hbm.at[pl.ds(off, C)], c_v)
            pltpu.sync_copy(p_hbm.at[pl.ds(off, C)], p_v)
            pltpu.sync_copy(mx_hbm.at[pl.ds(off, C)], mx_v)
            pltpu.sync_copy(my_hbm.at[pl.ds(off, C)], my_v)

            def j_body(v, _):
                o = v * _L
                bb = b_v[pl.ds(o, _L)]
                rr = r_v[pl.ds(o, _L)]
                mm = m_v[pl.ds(o, _L)]
                cc = c_v[pl.ds(o, _L)]
                j = jnp.where(mm > 0, (rr + 1) * nb + bb, bb)
                j_v[pl.ds(o, _L)] = j + cc * jrows
                return 0

            lax.fori_loop(0, C // _L, j_body, 0)

            cp_rows = pltpu.make_async_copy(table_hbm.at[j_v], rows_v, sem_a)
            cp_rows.start()
            cp_pts = pltpu.make_async_copy(pts_hbm.at[p_v], pts_v, sem_b)
            cp_pts.start()
            cp_rows.wait()
            cp_pts.wait()

            def c_body(v, _):
                o = v * _L
                row = o + iota
                m = [plsc.load_gather(rows_v, [row, jnp.full((_L,), k, jnp.int32)])
                     for k in range(12)]
                px = plsc.load_gather(pts_v, [row, jnp.full((_L,), 0, jnp.int32)])
                py = plsc.load_gather(pts_v, [row, jnp.full((_L,), 1, jnp.int32)])
                pz = plsc.load_gather(pts_v, [row, jnp.full((_L,), 2, jnp.int32)])
                xn = m[0] * px + m[1] * py + m[2] * pz + m[9]
                yn = m[3] * px + m[4] * py + m[5] * pz + m[10]
                zn = m[6] * px + m[7] * py + m[8] * pz + m[11]
                inv = 1.0 / zn
                mxv = mx_v[pl.ds(o, _L)]
                myv = my_v[pl.ds(o, _L)]
                plsc.store_scatter(out_v, [2 * row], xn * inv - mxv)
                plsc.store_scatter(out_v, [2 * row + 1], yn * inv - myv)
                return 0

            lax.fori_loop(0, C // _L, c_body, 0)
            pltpu.sync_copy(out_v, out_hbm.at[pl.ds(2 * off, 2 * C)])

        def chunk_body(g, _):
            do_chunk(base + g * C)
            return 0

        lax.fori_loop(0, n_full, chunk_body, 0)
        if rem > 0:
            @pl.when(wid == 0)
            def _():
                do_chunk(jnp.int32(tail1))

            @pl.when(wid == 1)
            def _():
                do_chunk(jnp.int32(tail2))

    return sc_kernel


def kernel(cam_indices, pt_indices, base_pose_indices, relative_pose_indices,
           is_relative, pixels_measured, base_quat, base_trans, rel_quat,
           rel_trans, points, intrinsics):
    n = cam_indices.shape[0]
    nb = base_quat.shape[0]
    nr = rel_quat.shape[0]
    ncam = intrinsics.shape[0]
    npts = points.shape[0]
    assert _NW == nr * ncam

    base1, rel1, pts1 = _build_planes(base_quat, base_trans, rel_quat,
                                      rel_trans, intrinsics, points)
    table, pts16 = _make_prep_kernel(nb, nr, ncam, npts)(base1, rel1, pts1)

    b_i = base_pose_indices.astype(jnp.int32)
    r_i = relative_pose_indices.astype(jnp.int32)
    m_i = is_relative.astype(jnp.int32)
    c_i = cam_indices.astype(jnp.int32)
    p_i = pt_indices.astype(jnp.int32)

    sc = _make_sc_kernel(n, nb, nr)
    return sc(b_i, r_i, m_i, c_i, p_i, pixels_measured[:, 0],
              pixels_measured[:, 1], table, pts16)


# speedup vs baseline: 74.2846x; 1.5989x over previous
import functools

import jax
import jax.numpy as jnp
from jax import lax
from jax.experimental import pallas as pl
from jax.experimental.pallas import tpu as pltpu
from jax.experimental.pallas import tpu_sc as plsc

_NC = 2
_NS = 16
_NW = _NC * _NS
_L = 16
_CHUNK = 512
_PCHUNK = 800

_SC_PARAMS = pltpu.CompilerParams(
    needs_layout_passes=False, use_tc_tiling_on_sc=False)


def _quat_planes(w, x, y, z):
    return [
        1.0 - 2.0 * (y * y + z * z), 2.0 * (x * y - w * z), 2.0 * (x * z + w * y),
        2.0 * (x * y + w * z), 1.0 - 2.0 * (x * x + z * z), 2.0 * (y * z - w * x),
        2.0 * (x * z - w * y), 2.0 * (y * z + w * x), 1.0 - 2.0 * (x * x + y * y),
    ]


def _planes_body(bq_ref, bt_ref, rq_ref, rt_ref, k_ref, pts_ref,
                 base_ref, rel_ref, pts1_ref):
    ncam = k_ref.shape[0] // 9
    nb = bq_ref.shape[1]
    nr = rq_ref.shape[0]
    npts = pts_ref.shape[1]

    for i in range(3):
        pts1_ref[pl.ds(i * npts, npts)] = pts_ref[i:i + 1, :].reshape(npts)

    w = bq_ref[0:1, :]
    x = bq_ref[1:2, :]
    y = bq_ref[2:3, :]
    z = bq_ref[3:4, :]
    inv = lax.rsqrt(w * w + x * x + y * y + z * z)
    w, x, y, z = w * inv, x * inv, y * inv, z * inv
    rb = _quat_planes(w, x, y, z)

    wr = rq_ref[:, 0:1]
    xr = rq_ref[:, 1:2]
    yr = rq_ref[:, 2:3]
    zr = rq_ref[:, 3:4]
    invr = lax.rsqrt(wr * wr + xr * xr + yr * yr + zr * zr)
    wr, xr, yr, zr = wr * invr, xr * invr, yr * invr, zr * invr
    rr = _quat_planes(wr, xr, yr, zr)

    rc = [None] * 9
    for i in range(3):
        for l in range(3):
            acc = rr[0 + l] * rb[3 * i + 0]
            acc = acc + rr[3 + l] * rb[3 * i + 1]
            acc = acc + rr[6 + l] * rb[3 * i + 2]
            rc[3 * i + l] = acc

    bt = [bt_ref[i:i + 1, :] for i in range(3)]
    rt = [rt_ref[:, i:i + 1] for i in range(3)]
    tc = [None] * 3
    for i in range(3):
        acc = bt[i] + rc[3 * i + 0] * rt[0]
        acc = acc + rc[3 * i + 1] * rt[1]
        acc = acc + rc[3 * i + 2] * rt[2]
        tc[i] = acc

    def wr_base(row, val):
        base_ref[pl.ds(row * nb, nb)] = val.reshape(nb)

    def wr_rel(row, val):
        for r in range(nr):
            rel_ref[pl.ds((row * nr + r) * nb, nb)] = val[r:r + 1, :].reshape(nb)

    for c in range(ncam):
        kk = [k_ref[9 * c + e] for e in range(9)]
        for i in range(3):
            for l in range(3):
                row = c * 12 + 3 * i + l
                wr_base(row, kk[3 * i] * rb[l] + kk[3 * i + 1] * rb[3 + l]
                        + kk[3 * i + 2] * rb[6 + l])
                wr_rel(row, kk[3 * i] * rc[l] + kk[3 * i + 1] * rc[3 + l]
                       + kk[3 * i + 2] * rc[6 + l])
            row = c * 12 + 9 + i
            wr_base(row, kk[3 * i] * bt[0] + kk[3 * i + 1] * bt[1]
                    + kk[3 * i + 2] * bt[2])
            wr_rel(row, kk[3 * i] * tc[0] + kk[3 * i + 1] * tc[1]
                   + kk[3 * i + 2] * tc[2])


def _build_planes(base_quat, base_trans, rel_quat, rel_trans, intrinsics,
                  points):
    nb = base_quat.shape[0]
    nr = rel_quat.shape[0]
    ncam = intrinsics.shape[0]
    npts = points.shape[0]
    return pl.pallas_call(
        _planes_body,
        out_shape=[
            jax.ShapeDtypeStruct((ncam * 12 * nb,), jnp.float32),
            jax.ShapeDtypeStruct((ncam * 12 * nr * nb,), jnp.float32),
            jax.ShapeDtypeStruct((3 * npts,), jnp.float32),
        ],
        in_specs=[
            pl.BlockSpec(memory_space=pltpu.VMEM),
            pl.BlockSpec(memory_space=pltpu.VMEM),
            pl.BlockSpec(memory_space=pltpu.VMEM),
            pl.BlockSpec(memory_space=pltpu.VMEM),
            pl.BlockSpec(memory_space=pltpu.SMEM),
            pl.BlockSpec(memory_space=pltpu.VMEM),
        ],
    )(base_quat.T, base_trans.T, rel_quat, rel_trans,
      intrinsics.reshape(ncam * 9), points.T)


def _make_prep_kernel(nb, nr, ncam, npts):
    jrows = (1 + nr) * nb
    n_psub = npts // _PCHUNK
    psub_per_w = -(-n_psub // _NW)
    assert npts % _PCHUNK == 0

    @functools.partial(
        pl.kernel,
        out_type=[
            jax.ShapeDtypeStruct((ncam * jrows, 16), jnp.float32),
            jax.ShapeDtypeStruct((npts, 16), jnp.float32),
        ],
        mesh=plsc.VectorSubcoreMesh(core_axis_name="c", subcore_axis_name="s",
                                    num_cores=_NC, num_subcores=_NS),
        compiler_params=_SC_PARAMS,
        scratch_types=[
            pltpu.VMEM((12, nb), jnp.float32),
            pltpu.VMEM((nb, 16), jnp.float32),
            pltpu.VMEM((3, _PCHUNK), jnp.float32),
            pltpu.VMEM((_PCHUNK, 16), jnp.float32),
        ],
    )
    def prep(base1_hbm, rel1_hbm, pts1_hbm, table_hbm, pts16_hbm,
             plane_v, seg_v, ppl_v, pbuf_v):
        wid = lax.axis_index("s") * _NC + lax.axis_index("c")
        iota = lax.iota(jnp.int32, _L)

        def interleave(src_v, dst_v, nrows, nent):
            def body(v, _):
                o = v * _L
                row = o + iota
                for e in range(nent):
                    plsc.store_scatter(
                        dst_v, [row, jnp.full((_L,), e, jnp.int32)],
                        src_v[e, pl.ds(o, _L)])
                return 0
            lax.fori_loop(0, nrows // _L, body, 0)

        c_rel = wid // nr
        r_rel = wid % nr
        for e in range(12):
            src = ((c_rel * 12 + e) * nr + r_rel) * nb
            pltpu.sync_copy(rel1_hbm.at[pl.ds(src, nb)], plane_v.at[e])
        interleave(plane_v, seg_v, nb, 12)
        dst_row = c_rel * jrows + (1 + r_rel) * nb
        pltpu.sync_copy(seg_v, table_hbm.at[pl.ds(dst_row, nb)])

        @pl.when(wid < ncam)
        def _():
            for e in range(12):
                src = (wid * 12 + e) * nb
                pltpu.sync_copy(base1_hbm.at[pl.ds(src, nb)], plane_v.at[e])
            interleave(plane_v, seg_v, nb, 12)
            pltpu.sync_copy(seg_v, table_hbm.at[pl.ds(wid * jrows, nb)])

        for s in range(psub_per_w):
            sub = wid * psub_per_w + s

            @pl.when(sub < n_psub)
            def _():
                row0 = sub * _PCHUNK
                for i in range(3):
                    pltpu.sync_copy(
                        pts1_hbm.at[pl.ds(i * npts + row0, _PCHUNK)],
                        ppl_v.at[i])
                interleave(ppl_v, pbuf_v, _PCHUNK, 3)
                pltpu.sync_copy(pbuf_v, pts16_hbm.at[pl.ds(row0, _PCHUNK)])

    return prep


def _make_sc_kernel(n, nb, nr):
    jrows = (1 + nr) * nb
    C = _CHUNK
    n_full = n // (C * _NW)
    w_per = n_full * C
    rem = n - w_per * _NW
    assert rem <= 2 * C and n % 8 == 0
    tail1 = w_per * _NW
    tail2 = n - C

    @functools.partial(
        pl.kernel,
        out_type=jax.ShapeDtypeStruct((2 * n,), jnp.float32),
        mesh=plsc.VectorSubcoreMesh(core_axis_name="c", subcore_axis_name="s",
                                    num_cores=_NC, num_subcores=_NS),
        compiler_params=_SC_PARAMS,
        scratch_types=[
            pltpu.VMEM((C,), jnp.int32),
            pltpu.VMEM((C,), jnp.int32),
            pltpu.VMEM((C,), jnp.int32),
            pltpu.VMEM((C,), jnp.int32),
            pltpu.VMEM((C,), jnp.int32),
            pltpu.VMEM((C,), jnp.int32),
            pltpu.VMEM((C,), jnp.float32),
            pltpu.VMEM((C,), jnp.float32),
            pltpu.VMEM((C, 16), jnp.float32),
            pltpu.VMEM((C, 16), jnp.float32),
            pltpu.VMEM((2 * C,), jnp.float32),
            pltpu.SemaphoreType.DMA,
            pltpu.SemaphoreType.DMA,
        ],
    )
    def sc_kernel(b_hbm, r_hbm, m_hbm, c_hbm, p_hbm, mx_hbm, my_hbm,
                  table_hbm, pts_hbm, out_hbm, b_v, r_v, m_v, c_v, p_v, j_v,
                  mx_v, my_v, rows_v, pts_v, out_v, sem_a, sem_b):
        wid = lax.axis_index("s") * _NC + lax.axis_index("c")
        base = wid * w_per
        iota = lax.iota(jnp.int32, _L)

        def do_chunk(off):
            cps = [
                pltpu.make_async_copy(b_hbm.at[pl.ds(off, C)], b_v, sem_a),
                pltpu.make_async_copy(r_hbm.at[pl.ds(off, C)], r_v, sem_a),
                pltpu.make_async_copy(m_hbm.at[pl.ds(off, C)], m_v, sem_a),
                pltpu.make_async_copy(c_hbm.at[pl.ds(off, C)], c_v, sem_a),
                pltpu.make_async_copy(p_hbm.at[pl.ds(off, C)], p_v, sem_a),
                pltpu.make_async_copy(mx_hbm.at[pl.ds(off, C)], mx_v, sem_a),
                pltpu.make_async_copy(my_hbm.at[pl.ds(off, C)], my_v, sem_a),
            ]
            for cp in cps:
                cp.start()
            for cp in cps:
                cp.wait()

            def j_body(v, _):
                o = v * _L
                bb = b_v[pl.ds(o, _L)]
                rr = r_v[pl.ds(o, _L)]
                mm = m_v[pl.ds(o, _L)]
                cc = c_v[pl.ds(o, _L)]
                j = jnp.where(mm > 0, (rr + 1) * nb + bb, bb)
                j_v[pl.ds(o, _L)] = j + cc * jrows
                return 0

            lax.fori_loop(0, C // _L, j_body, 0)

            cp_rows = pltpu.make_async_copy(table_hbm.at[j_v], rows_v, sem_a)
            cp_rows.start()
            cp_pts = pltpu.make_async_copy(pts_hbm.at[p_v], pts_v, sem_b)
            cp_pts.start()
            cp_rows.wait()
            cp_pts.wait()

            def c_body(v, _):
                o = v * _L
                row = o + iota
                m = [plsc.load_gather(rows_v, [row, jnp.full((_L,), k, jnp.int32)])
                     for k in range(12)]
                px = plsc.load_gather(pts_v, [row, jnp.full((_L,), 0, jnp.int32)])
                py = plsc.load_gather(pts_v, [row, jnp.full((_L,), 1, jnp.int32)])
                pz = plsc.load_gather(pts_v, [row, jnp.full((_L,), 2, jnp.int32)])
                xn = m[0] * px + m[1] * py + m[2] * pz + m[9]
                yn = m[3] * px + m[4] * py + m[5] * pz + m[10]
                zn = m[6] * px + m[7] * py + m[8] * pz + m[11]
                inv = 1.0 / zn
                mxv = mx_v[pl.ds(o, _L)]
                myv = my_v[pl.ds(o, _L)]
                plsc.store_scatter(out_v, [2 * row], xn * inv - mxv)
                plsc.store_scatter(out_v, [2 * row + 1], yn * inv - myv)
                return 0

            lax.fori_loop(0, C // _L, c_body, 0)
            pltpu.sync_copy(out_v, out_hbm.at[pl.ds(2 * off, 2 * C)])

        def chunk_body(g, _):
            do_chunk(base + g * C)
            return 0

        lax.fori_loop(0, n_full, chunk_body, 0)
        if rem > 0:
            @pl.when(wid == 0)
            def _():
                do_chunk(jnp.int32(tail1))

            @pl.when(wid == 1)
            def _():
                do_chunk(jnp.int32(tail2))

    return sc_kernel


def kernel(cam_indices, pt_indices, base_pose_indices, relative_pose_indices,
           is_relative, pixels_measured, base_quat, base_trans, rel_quat,
           rel_trans, points, intrinsics):
    n = cam_indices.shape[0]
    nb = base_quat.shape[0]
    nr = rel_quat.shape[0]
    ncam = intrinsics.shape[0]
    npts = points.shape[0]
    assert _NW == nr * ncam

    base1, rel1, pts1 = _build_planes(base_quat, base_trans, rel_quat,
                                      rel_trans, intrinsics, points)
    table, pts16 = _make_prep_kernel(nb, nr, ncam, npts)(base1, rel1, pts1)

    b_i = base_pose_indices.astype(jnp.int32)
    r_i = relative_pose_indices.astype(jnp.int32)
    m_i = is_relative.astype(jnp.int32)
    c_i = cam_indices.astype(jnp.int32)
    p_i = pt_indices.astype(jnp.int32)

    sc = _make_sc_kernel(n, nb, nr)
    return sc(b_i, r_i, m_i, c_i, p_i, pixels_measured[:, 0],
              pixels_measured[:, 1], table, pts16)
